# baseline (speedup 1.0000x reference)
import functools

import jax
import jax.numpy as jnp
from jax.experimental import pallas as pl
from jax.experimental.pallas import tpu as pltpu

K = 50


def _scalar_at(arr2d, flat_idx):
    r, c = arr2d.shape
    ri = jax.lax.broadcasted_iota(jnp.int32, (r, c), 0)
    ci = jax.lax.broadcasted_iota(jnp.int32, (r, c), 1)
    want_r = flat_idx // c
    want_c = flat_idx % c
    return jnp.sum(jnp.where((ri == want_r) & (ci == want_c), arr2d, 0.0))


def _topk_weighted(sim, q, avgrow):
    s = jnp.where((q > 0.0) & (sim > 0.0), sim, 0.0)
    sb = jax.lax.bitcast_convert_type(s, jnp.int32)

    def body(_, lohi):
        lo, hi = lohi
        mid = (lo + hi + 1) // 2
        cnt = jnp.sum((sb >= mid).astype(jnp.int32))
        ok = cnt >= K
        return (jnp.where(ok, mid, lo), jnp.where(ok, hi, mid))

    lo, _ = jax.lax.fori_loop(
        0, 30, body, (jnp.int32(0), jnp.int32(1 << 30)), unroll=2)
    sel = sb >= lo
    ssum = jnp.sum(jnp.where(sel, s, 0.0))
    wsum = jnp.sum(jnp.where(sel, s * (q - avgrow), 0.0))
    return wsum / (ssum + 1e-8)


def _body(uid_ref, iid_ref, tid_ref,
          item_sim_ref, qos_row_ref, item_avg_ref,
          user_sim_ref, qos_col_ref, user_avg_ref,
          tsum_ref, tcnt_ref, w_ref, out_ref):
    b = pl.program_id(0)
    i_id = iid_ref[b]
    u_id = uid_ref[b]

    sim_i = item_sim_ref[0]
    qrow = qos_row_ref[0, 0]
    iavg = item_avg_ref[0]
    sim_u = user_sim_ref[0]
    qcol = qos_col_ref[0, 0]
    uavg = user_avg_ref[0]

    curr = _scalar_at(qrow, i_id)
    curr_nz = (curr > 0.0).astype(jnp.float32)
    lc = tsum_ref.shape[-1]
    ts = tsum_ref[0, 0] - curr
    tc = tcnt_ref[0, 0] - curr_nz
    lane = i_id % lc
    sum_o = _scalar_at(ts, lane)
    cnt_o = _scalar_at(tc, lane)
    p_t = jnp.where(cnt_o > 0.0, sum_o / jnp.maximum(cnt_o, 1.0), 0.0)

    avg_u = _scalar_at(uavg, u_id)
    p_u = avg_u + _topk_weighted(sim_u, qcol, uavg)

    avg_i = _scalar_at(iavg, i_id)
    p_i = avg_i + _topk_weighted(sim_i, qrow, iavg)

    val = (w_ref[0, 0] * p_t + w_ref[0, 1] * p_u + w_ref[0, 2] * p_i)
    out_ref[0] = jnp.full((1, 128), val, dtype=jnp.float32)


@functools.partial(jax.jit, static_argnames=())
def kernel(qos, item_avg, user_avg, item_sim_agg, user_sim_agg,
           total_sum, total_cnt, weights, user_id, item_id, time_id):
    T, U, I = qos.shape
    B = user_id.shape[0]
    lc = min(128, I)

    qos_r = qos.reshape(T, U, 8, I // 8)
    qos_t = qos.transpose(0, 2, 1).reshape(T, I, 4, U // 4)
    item_sim_r = item_sim_agg.reshape(I, 8, I // 8)
    user_sim_r = user_sim_agg.reshape(U, 4, U // 4)
    item_avg_r = item_avg.reshape(T, 8, I // 8)
    user_avg_r = user_avg.reshape(T, 4, U // 4)
    tsum_r = total_sum.reshape(U, I // lc, 1, lc)
    tcnt_r = total_cnt.reshape(U, I // lc, 1, lc)
    w2d = weights.reshape(1, 3)

    grid_spec = pltpu.PrefetchScalarGridSpec(
        num_scalar_prefetch=3,
        grid=(B,),
        in_specs=[
            pl.BlockSpec((1, 8, I // 8), lambda b, u, i, t: (i[b], 0, 0)),
            pl.BlockSpec((1, 1, 8, I // 8), lambda b, u, i, t: (t[b], u[b], 0, 0)),
            pl.BlockSpec((1, 8, I // 8), lambda b, u, i, t: (t[b], 0, 0)),
            pl.BlockSpec((1, 4, U // 4), lambda b, u, i, t: (u[b], 0, 0)),
            pl.BlockSpec((1, 1, 4, U // 4), lambda b, u, i, t: (t[b], i[b], 0, 0)),
            pl.BlockSpec((1, 4, U // 4), lambda b, u, i, t: (t[b], 0, 0)),
            pl.BlockSpec((1, 1, 1, lc), lambda b, u, i, t: (u[b], i[b] // lc, 0, 0)),
            pl.BlockSpec((1, 1, 1, lc), lambda b, u, i, t: (u[b], i[b] // lc, 0, 0)),
            pl.BlockSpec((1, 3), lambda b, u, i, t: (0, 0)),
        ],
        out_specs=pl.BlockSpec((1, 1, 128), lambda b, u, i, t: (b, 0, 0)),
    )
    out = pl.pallas_call(
        _body,
        grid_spec=grid_spec,
        out_shape=jax.ShapeDtypeStruct((B, 1, 128), jnp.float32),
    )(user_id, item_id, time_id,
      item_sim_r, qos_r, item_avg_r, user_sim_r, qos_t, user_avg_r,
      tsum_r, tcnt_r, w2d)
    return out[:, 0, 0]


# speedup vs baseline: 16.2233x; 16.2233x over previous
import functools

import jax
import jax.numpy as jnp
from jax import lax
from jax.experimental import pallas as pl
from jax.experimental.pallas import tpu as pltpu
from jax.experimental.pallas import tpu_sc as plsc

K = 50
NW = 32
CH = 8


def _make_sc_kernel(T, U, I, B):
    BPW = B // NW
    NCH = BPW // CH
    NG = BPW // 16

    mesh = plsc.VectorSubcoreMesh(core_axis_name="c", subcore_axis_name="s")

    scratch = [
        pltpu.VMEM((BPW,), jnp.int32),
        pltpu.VMEM((BPW,), jnp.int32),
        pltpu.VMEM((BPW,), jnp.int32),
        pltpu.VMEM((BPW,), jnp.int32),
        pltpu.VMEM((BPW,), jnp.int32),
        pltpu.VMEM((BPW,), jnp.int32),
        pltpu.VMEM((BPW, 16), jnp.float32),
        pltpu.VMEM((BPW,), jnp.float32),
        pltpu.VMEM((BPW,), jnp.float32),
        pltpu.VMEM((BPW,), jnp.float32),
        pltpu.VMEM((T * I,), jnp.float32),
        pltpu.VMEM((T * U,), jnp.float32),
        pltpu.VMEM((16,), jnp.float32),
        pltpu.VMEM((CH, I), jnp.float32),
        pltpu.VMEM((CH, I), jnp.float32),
        pltpu.VMEM((CH, U), jnp.float32),
        pltpu.VMEM((CH, U), jnp.float32),
        pltpu.VMEM((I + 16,), jnp.float32),
        pltpu.VMEM((I + 16,), jnp.float32),
        pltpu.SemaphoreType.DMA,
    ]

    @functools.partial(
        pl.kernel, mesh=mesh,
        out_type=jax.ShapeDtypeStruct((B,), jnp.float32),
        compiler_params=pltpu.CompilerParams(needs_layout_passes=False,
                                             use_tc_tiling_on_sc=False),
        scratch_types=scratch,
    )
    def sc_kernel(qrows_hbm, qcols_hbm, isim_hbm, usim_hbm,
                  tsg_hbm, tcg_hbm,
                  iavg_hbm, uavg_hbm, w_hbm, uid_hbm, iid_hbm, tid_hbm,
                  out_hbm,
                  uid_v, iid_v, tid_v,
                  idx_qrow, idx_qcol, idx_ts,
                  tmp_g, ts_v, tc_v, out_v,
                  iavg_v, uavg_v, w_v,
                  sim_buf, qrow_buf, col_buf, usim_buf,
                  cand_s, cand_w, sem):
        wid = lax.axis_index("s") * 2 + lax.axis_index("c")
        base = wid * BPW

        pltpu.sync_copy(uid_hbm.at[pl.ds(base, BPW)], uid_v)
        pltpu.sync_copy(iid_hbm.at[pl.ds(base, BPW)], iid_v)
        pltpu.sync_copy(tid_hbm.at[pl.ds(base, BPW)], tid_v)
        pltpu.sync_copy(iavg_hbm, iavg_v)
        pltpu.sync_copy(uavg_hbm, uavg_v)
        pltpu.sync_copy(w_hbm, w_v)

        lane = lax.iota(jnp.int32, 16)

        for g in range(NG):
            sl = pl.ds(g * 16, 16)
            u = uid_v[sl]
            i = iid_v[sl]
            t = tid_v[sl]
            idx_qrow[sl] = t * U + u
            idx_qcol[sl] = t * I + i
            idx_ts[sl] = (u * I + i) // 16

        for tbl, outr in ((tsg_hbm, ts_v), (tcg_hbm, tc_v)):
            pltpu.async_copy(tbl.at[idx_ts], tmp_g, sem).wait()
            for g in range(NG):
                sl = pl.ds(g * 16, 16)
                u = uid_v[sl]
                i = iid_v[sl]
                rows = lane + g * 16
                outr[sl] = plsc.load_gather(tmp_g, [rows, (u * I + i) % 16])

        w16 = w_v[pl.ds(0, 16)]
        w0 = jnp.sum(jnp.where(lane == 0, w16, 0.0))
        w1 = jnp.sum(jnp.where(lane == 1, w16, 0.0))
        w2 = jnp.sum(jnp.where(lane == 2, w16, 0.0))

        zf = jnp.zeros((16,), jnp.float32)
        zi = jnp.zeros((16,), jnp.int32)

        def branch(load_sim, load_q, load_avg, n):
            def p_a(j, off):
                sim = load_sim(j)
                q = load_q(j)
                av = load_avg(j)
                m = (q > 0.0) & (sim > 0.0)
                plsc.store_compressed(cand_s.at[pl.ds(off, 16)], sim, mask=m)
                plsc.store_compressed(cand_w.at[pl.ds(off, 16)], q - av,
                                      mask=m)
                return off + jnp.sum(m.astype(jnp.int32))

            off = lax.fori_loop(0, n // 16, p_a, jnp.int32(0))
            cand_s[pl.ds(off, 16)] = zf
            cand_w[pl.ds(off, 16)] = zf
            nv = off // 16 + 1

            def p_b(it, lohi):
                lo, hi = lohi
                mid = (lo + hi + 1) // 2

                def cb(v, acc):
                    bits = plsc.bitcast(cand_s[pl.ds(v * 16, 16)], jnp.int32)
                    return acc + jnp.where(bits >= mid, 1, 0)

                cnt = jnp.sum(lax.fori_loop(0, nv, cb, zi))
                ok = cnt >= K
                return (jnp.where(ok, mid, lo), jnp.where(ok, hi, mid))

            tau, _ = lax.fori_loop(
                0, 30, p_b, (jnp.int32(0), jnp.int32(1 << 30)))

            def p_c(v, carry):
                ssum, wsum = carry
                s = cand_s[pl.ds(v * 16, 16)]
                w = cand_w[pl.ds(v * 16, 16)]
                sel = plsc.bitcast(s, jnp.int32) >= tau
                return (ssum + jnp.where(sel, s, 0.0),
                        wsum + jnp.where(sel, s * w, 0.0))

            ssum, wsum = lax.fori_loop(0, nv, p_c, (zf, zf))
            return jnp.max((zf + jnp.sum(wsum))
                           / (zf + jnp.sum(ssum) + 1e-8))

        def chunk_body(c, carry):
            cb = c * CH
            h1 = pltpu.async_copy(isim_hbm.at[iid_v.at[pl.ds(cb, CH)]],
                                  sim_buf, sem)
            h2 = pltpu.async_copy(qrows_hbm.at[idx_qrow.at[pl.ds(cb, CH)]],
                                  qrow_buf, sem)
            h3 = pltpu.async_copy(qcols_hbm.at[idx_qcol.at[pl.ds(cb, CH)]],
                                  col_buf, sem)
            h4 = pltpu.async_copy(usim_hbm.at[uid_v.at[pl.ds(cb, CH)]],
                                  usim_buf, sem)
            h1.wait()
            h2.wait()
            h3.wait()
            h4.wait()
            for r in range(CH):
                s_pos = cb + r
                spl = jnp.zeros((16,), jnp.int32) + s_pos
                t_s = jnp.max(plsc.load_gather(tid_v, [spl]))
                u_s = jnp.max(plsc.load_gather(uid_v, [spl]))
                i_s = jnp.max(plsc.load_gather(iid_v, [spl]))
                ib = t_s * I
                ub = t_s * U
                p_i = branch(
                    lambda j: sim_buf[r, pl.ds(j * 16, 16)],
                    lambda j: qrow_buf[r, pl.ds(j * 16, 16)],
                    lambda j: iavg_v[pl.ds(ib + j * 16, 16)], I)
                p_u = branch(
                    lambda j: usim_buf[r, pl.ds(j * 16, 16)],
                    lambda j: col_buf[r, pl.ds(j * 16, 16)],
                    lambda j: uavg_v[pl.ds(ub + j * 16, 16)], U)
                rspl = jnp.zeros((16,), jnp.int32) + r
                curr = jnp.max(plsc.load_gather(qrow_buf,
                                                [rspl, zi + i_s]))
                ts_s = jnp.max(plsc.load_gather(ts_v, [spl]))
                tc_s = jnp.max(plsc.load_gather(tc_v, [spl]))
                nz = jnp.where(curr > 0.0, 1.0, 0.0)
                sum_o = ts_s - curr
                cnt_o = tc_s - nz
                qt = jnp.max((zf + sum_o) / (zf + jnp.maximum(cnt_o, 1.0)))
                p_t = jnp.where(cnt_o > 0.0, qt, 0.0)
                bi_s = jnp.max(plsc.load_gather(iavg_v, [zi + (ib + i_s)]))
                bu_s = jnp.max(plsc.load_gather(uavg_v, [zi + (ub + u_s)]))
                val = w0 * p_t + w1 * (bu_s + p_u) + w2 * (bi_s + p_i)
                plsc.store_scatter(out_v, [spl], zf + val, mask=(lane == 0))
            return carry

        lax.fori_loop(0, NCH, chunk_body, jnp.int32(0))
        pltpu.sync_copy(out_v, out_hbm.at[pl.ds(base, BPW)])

    return sc_kernel


def kernel(qos, item_avg, user_avg, item_sim_agg, user_sim_agg,
           total_sum, total_cnt, weights, user_id, item_id, time_id):
    T, U, I = qos.shape
    B = user_id.shape[0]

    qrows = qos.reshape(T * U, I)
    qcols = qos.transpose(0, 2, 1).reshape(T * I, U)
    tsg = total_sum.reshape(U * I // 16, 16)
    tcg = total_cnt.reshape(U * I // 16, 16)
    iavg_flat = item_avg.reshape(T * I)
    uavg_flat = user_avg.reshape(T * U)
    w_pad = jnp.zeros((16,), jnp.float32).at[:3].set(weights)

    sc = _make_sc_kernel(T, U, I, B)
    return sc(qrows, qcols, item_sim_agg, user_sim_agg, tsg, tcg,
              iavg_flat, uavg_flat, w_pad, user_id, item_id, time_id)


# speedup vs baseline: 28.5562x; 1.7602x over previous
import functools

import jax
import jax.numpy as jnp
from jax import lax
from jax.experimental import pallas as pl
from jax.experimental.pallas import tpu as pltpu
from jax.experimental.pallas import tpu_sc as plsc

K = 50
NW = 32
CH = 8


def _make_sc_kernel(T, U, I, B):
    BPW = B // NW
    NCH = BPW // CH
    NG = BPW // 16

    mesh = plsc.VectorSubcoreMesh(core_axis_name="c", subcore_axis_name="s")

    scratch = [
        pltpu.VMEM((BPW,), jnp.int32),
        pltpu.VMEM((BPW,), jnp.int32),
        pltpu.VMEM((BPW,), jnp.int32),
        pltpu.VMEM((BPW,), jnp.int32),
        pltpu.VMEM((BPW,), jnp.int32),
        pltpu.VMEM((BPW,), jnp.int32),
        pltpu.VMEM((BPW, 16), jnp.float32),
        pltpu.VMEM((BPW,), jnp.float32),
        pltpu.VMEM((BPW,), jnp.float32),
        pltpu.VMEM((BPW,), jnp.float32),
        pltpu.VMEM((T * I,), jnp.float32),
        pltpu.VMEM((T * U,), jnp.float32),
        pltpu.VMEM((16,), jnp.float32),
        pltpu.VMEM((CH, I), jnp.float32),
        pltpu.VMEM((CH, I), jnp.float32),
        pltpu.VMEM((CH, U), jnp.float32),
        pltpu.VMEM((CH, U), jnp.float32),
        pltpu.VMEM((I + 16,), jnp.float32),
        pltpu.VMEM((I + 16,), jnp.float32),
        pltpu.SemaphoreType.DMA,
    ]

    @functools.partial(
        pl.kernel, mesh=mesh,
        out_type=jax.ShapeDtypeStruct((B,), jnp.float32),
        compiler_params=pltpu.CompilerParams(needs_layout_passes=False,
                                             use_tc_tiling_on_sc=False),
        scratch_types=scratch,
    )
    def sc_kernel(qrows_hbm, qcols_hbm, isim_hbm, usim_hbm,
                  tsg_hbm, tcg_hbm,
                  iavg_hbm, uavg_hbm, w_hbm, uid_hbm, iid_hbm, tid_hbm,
                  out_hbm,
                  uid_v, iid_v, tid_v,
                  idx_qrow, idx_qcol, idx_ts,
                  tmp_g, ts_v, tc_v, out_v,
                  iavg_v, uavg_v, w_v,
                  sim_buf, qrow_buf, col_buf, usim_buf,
                  cand_s, cand_w, sem):
        wid = lax.axis_index("s") * 2 + lax.axis_index("c")
        base = wid * BPW

        pltpu.sync_copy(uid_hbm.at[pl.ds(base, BPW)], uid_v)
        pltpu.sync_copy(iid_hbm.at[pl.ds(base, BPW)], iid_v)
        pltpu.sync_copy(tid_hbm.at[pl.ds(base, BPW)], tid_v)
        pltpu.sync_copy(iavg_hbm, iavg_v)
        pltpu.sync_copy(uavg_hbm, uavg_v)
        pltpu.sync_copy(w_hbm, w_v)

        lane = lax.iota(jnp.int32, 16)

        for g in range(NG):
            sl = pl.ds(g * 16, 16)
            u = uid_v[sl]
            i = iid_v[sl]
            t = tid_v[sl]
            idx_qrow[sl] = t * U + u
            idx_qcol[sl] = t * I + i
            idx_ts[sl] = (u * I + i) // 16

        for tbl, outr in ((tsg_hbm, ts_v), (tcg_hbm, tc_v)):
            pltpu.async_copy(tbl.at[idx_ts], tmp_g, sem).wait()
            for g in range(NG):
                sl = pl.ds(g * 16, 16)
                u = uid_v[sl]
                i = iid_v[sl]
                rows = lane + g * 16
                outr[sl] = plsc.load_gather(tmp_g, [rows, (u * I + i) % 16])

        w16 = w_v[pl.ds(0, 16)]
        w0 = jnp.sum(jnp.where(lane == 0, w16, 0.0))
        w1 = jnp.sum(jnp.where(lane == 1, w16, 0.0))
        w2 = jnp.sum(jnp.where(lane == 2, w16, 0.0))

        zf = jnp.zeros((16,), jnp.float32)
        zi = jnp.zeros((16,), jnp.int32)

        def branch(load_sim, load_q, load_avg, n):
            def p_a(j, off):
                sim = load_sim(j)
                q = load_q(j)
                av = load_avg(j)
                m = (q > 0.0) & (sim > 0.0)
                plsc.store_compressed(cand_s.at[pl.ds(off, 16)], sim, mask=m)
                plsc.store_compressed(cand_w.at[pl.ds(off, 16)], q - av,
                                      mask=m)
                return off + jnp.sum(m.astype(jnp.int32))

            off = lax.fori_loop(0, n // 16, p_a, jnp.int32(0), unroll=4)
            cand_s[pl.ds(off, 16)] = zf
            cand_w[pl.ds(off, 16)] = zf
            nv = off // 16 + 1

            def p_b(it, lohi):
                lo, hi = lohi
                mid = (lo + hi + 1) // 2

                def cb(v, acc):
                    bits = plsc.bitcast(cand_s[pl.ds(v * 16, 16)], jnp.int32)
                    return acc + jnp.where(bits >= mid, 1, 0)

                cnt = jnp.sum(plsc.parallel_loop(0, nv, unroll=8,
                                                 carry=zi)(cb))
                ok = cnt >= K
                return (jnp.where(ok, mid, lo), jnp.where(ok, hi, mid))

            tau, _ = lax.fori_loop(
                0, 30, p_b, (jnp.int32(0), jnp.int32(1 << 30)))

            def p_c(v, carry):
                ssum, wsum = carry
                s = cand_s[pl.ds(v * 16, 16)]
                w = cand_w[pl.ds(v * 16, 16)]
                sel = plsc.bitcast(s, jnp.int32) >= tau
                return (ssum + jnp.where(sel, s, 0.0),
                        wsum + jnp.where(sel, s * w, 0.0))

            ssum, wsum = plsc.parallel_loop(0, nv, unroll=4,
                                            carry=(zf, zf))(p_c)
            return jnp.max((zf + jnp.sum(wsum))
                           / (zf + jnp.sum(ssum) + 1e-8))

        def chunk_body(c, carry):
            cb = c * CH
            h1 = pltpu.async_copy(isim_hbm.at[iid_v.at[pl.ds(cb, CH)]],
                                  sim_buf, sem)
            h2 = pltpu.async_copy(qrows_hbm.at[idx_qrow.at[pl.ds(cb, CH)]],
                                  qrow_buf, sem)
            h3 = pltpu.async_copy(qcols_hbm.at[idx_qcol.at[pl.ds(cb, CH)]],
                                  col_buf, sem)
            h4 = pltpu.async_copy(usim_hbm.at[uid_v.at[pl.ds(cb, CH)]],
                                  usim_buf, sem)
            h1.wait()
            h2.wait()
            h3.wait()
            h4.wait()
            for r in range(CH):
                s_pos = cb + r
                spl = jnp.zeros((16,), jnp.int32) + s_pos
                t_s = jnp.max(plsc.load_gather(tid_v, [spl]))
                u_s = jnp.max(plsc.load_gather(uid_v, [spl]))
                i_s = jnp.max(plsc.load_gather(iid_v, [spl]))
                ib = t_s * I
                ub = t_s * U
                p_i = branch(
                    lambda j: sim_buf[r, pl.ds(j * 16, 16)],
                    lambda j: qrow_buf[r, pl.ds(j * 16, 16)],
                    lambda j: iavg_v[pl.ds(ib + j * 16, 16)], I)
                p_u = branch(
                    lambda j: usim_buf[r, pl.ds(j * 16, 16)],
                    lambda j: col_buf[r, pl.ds(j * 16, 16)],
                    lambda j: uavg_v[pl.ds(ub + j * 16, 16)], U)
                rspl = jnp.zeros((16,), jnp.int32) + r
                curr = jnp.max(plsc.load_gather(qrow_buf,
                                                [rspl, zi + i_s]))
                ts_s = jnp.max(plsc.load_gather(ts_v, [spl]))
                tc_s = jnp.max(plsc.load_gather(tc_v, [spl]))
                nz = jnp.where(curr > 0.0, 1.0, 0.0)
                sum_o = ts_s - curr
                cnt_o = tc_s - nz
                qt = jnp.max((zf + sum_o) / (zf + jnp.maximum(cnt_o, 1.0)))
                p_t = jnp.where(cnt_o > 0.0, qt, 0.0)
                bi_s = jnp.max(plsc.load_gather(iavg_v, [zi + (ib + i_s)]))
                bu_s = jnp.max(plsc.load_gather(uavg_v, [zi + (ub + u_s)]))
                val = w0 * p_t + w1 * (bu_s + p_u) + w2 * (bi_s + p_i)
                plsc.store_scatter(out_v, [spl], zf + val, mask=(lane == 0))
            return carry

        lax.fori_loop(0, NCH, chunk_body, jnp.int32(0))
        pltpu.sync_copy(out_v, out_hbm.at[pl.ds(base, BPW)])

    return sc_kernel


def kernel(qos, item_avg, user_avg, item_sim_agg, user_sim_agg,
           total_sum, total_cnt, weights, user_id, item_id, time_id):
    T, U, I = qos.shape
    B = user_id.shape[0]

    qrows = qos.reshape(T * U, I)
    qcols = qos.transpose(0, 2, 1).reshape(T * I, U)
    tsg = total_sum.reshape(U * I // 16, 16)
    tcg = total_cnt.reshape(U * I // 16, 16)
    iavg_flat = item_avg.reshape(T * I)
    uavg_flat = user_avg.reshape(T * U)
    w_pad = jnp.zeros((16,), jnp.float32).at[:3].set(weights)

    sc = _make_sc_kernel(T, U, I, B)
    return sc(qrows, qcols, item_sim_agg, user_sim_agg, tsg, tcg,
              iavg_flat, uavg_flat, w_pad, user_id, item_id, time_id)


# speedup vs baseline: 31.7735x; 1.1127x over previous
import functools

import jax
import jax.numpy as jnp
from jax import lax
from jax.experimental import pallas as pl
from jax.experimental.pallas import tpu as pltpu
from jax.experimental.pallas import tpu_sc as plsc

K = 50
NW = 32
CH = 8


def _make_sc_kernel(T, U, I, B):
    BPW = B // NW
    NCH = BPW // CH
    NG = BPW // 16

    mesh = plsc.VectorSubcoreMesh(core_axis_name="c", subcore_axis_name="s")

    scratch = [
        pltpu.VMEM((BPW,), jnp.int32),
        pltpu.VMEM((BPW,), jnp.int32),
        pltpu.VMEM((BPW,), jnp.int32),
        pltpu.VMEM((BPW,), jnp.int32),
        pltpu.VMEM((BPW,), jnp.int32),
        pltpu.VMEM((BPW,), jnp.int32),
        pltpu.VMEM((BPW, 16), jnp.float32),
        pltpu.VMEM((BPW,), jnp.float32),
        pltpu.VMEM((BPW,), jnp.float32),
        pltpu.VMEM((BPW,), jnp.float32),
        pltpu.VMEM((T * I,), jnp.float32),
        pltpu.VMEM((T * U,), jnp.float32),
        pltpu.VMEM((16,), jnp.float32),
        pltpu.VMEM((CH, I), jnp.float32),
        pltpu.VMEM((CH, I), jnp.float32),
        pltpu.VMEM((CH, U), jnp.float32),
        pltpu.VMEM((CH, U), jnp.float32),
        pltpu.VMEM((I + 16,), jnp.float32),
        pltpu.VMEM((I + 16,), jnp.float32),
        pltpu.VMEM((1024,), jnp.int32),
        pltpu.VMEM((528,), jnp.float32),
        pltpu.SemaphoreType.DMA,
    ]

    @functools.partial(
        pl.kernel, mesh=mesh,
        out_type=jax.ShapeDtypeStruct((B,), jnp.float32),
        compiler_params=pltpu.CompilerParams(needs_layout_passes=False,
                                             use_tc_tiling_on_sc=False),
        scratch_types=scratch,
    )
    def sc_kernel(qrows_hbm, qcols_hbm, isim_hbm, usim_hbm,
                  tsg_hbm, tcg_hbm,
                  iavg_hbm, uavg_hbm, w_hbm, uid_hbm, iid_hbm, tid_hbm,
                  out_hbm,
                  uid_v, iid_v, tid_v,
                  idx_qrow, idx_qcol, idx_ts,
                  tmp_g, ts_v, tc_v, out_v,
                  iavg_v, uavg_v, w_v,
                  sim_buf, qrow_buf, col_buf, usim_buf,
                  cand_s, cand_w, hist, cand2, sem):
        wid = lax.axis_index("s") * 2 + lax.axis_index("c")
        base = wid * BPW

        pltpu.sync_copy(uid_hbm.at[pl.ds(base, BPW)], uid_v)
        pltpu.sync_copy(iid_hbm.at[pl.ds(base, BPW)], iid_v)
        pltpu.sync_copy(tid_hbm.at[pl.ds(base, BPW)], tid_v)
        pltpu.sync_copy(iavg_hbm, iavg_v)
        pltpu.sync_copy(uavg_hbm, uavg_v)
        pltpu.sync_copy(w_hbm, w_v)

        lane = lax.iota(jnp.int32, 16)

        for g in range(NG):
            sl = pl.ds(g * 16, 16)
            u = uid_v[sl]
            i = iid_v[sl]
            t = tid_v[sl]
            idx_qrow[sl] = t * U + u
            idx_qcol[sl] = t * I + i
            idx_ts[sl] = (u * I + i) // 16

        for tbl, outr in ((tsg_hbm, ts_v), (tcg_hbm, tc_v)):
            pltpu.async_copy(tbl.at[idx_ts], tmp_g, sem).wait()
            for g in range(NG):
                sl = pl.ds(g * 16, 16)
                u = uid_v[sl]
                i = iid_v[sl]
                rows = lane + g * 16
                outr[sl] = plsc.load_gather(tmp_g, [rows, (u * I + i) % 16])

        w16 = w_v[pl.ds(0, 16)]
        w0 = jnp.sum(jnp.where(lane == 0, w16, 0.0))
        w1 = jnp.sum(jnp.where(lane == 1, w16, 0.0))
        w2 = jnp.sum(jnp.where(lane == 2, w16, 0.0))

        zf = jnp.zeros((16,), jnp.float32)
        zi = jnp.zeros((16,), jnp.int32)

        def branch(load_sim, load_q, load_avg, n):
            def p_a(j, off):
                sim = load_sim(j)
                q = load_q(j)
                av = load_avg(j)
                m = (q > 0.0) & (sim > 0.0)
                plsc.store_compressed(cand_s.at[pl.ds(off, 16)], sim, mask=m)
                plsc.store_compressed(cand_w.at[pl.ds(off, 16)], q - av,
                                      mask=m)
                return off + jnp.sum(m.astype(jnp.int32))

            off = lax.fori_loop(0, n // 16, p_a, jnp.int32(0), unroll=4)
            cand_s[pl.ds(off, 16)] = zf
            cand_w[pl.ds(off, 16)] = zf
            nv = off // 16 + 1

            NBV = 64

            @plsc.parallel_loop(0, NBV, unroll=8)
            def _clear(v):
                hist[pl.ds(v * 16, 16)] = zi

            @plsc.parallel_loop(0, nv, unroll=4)
            def _hist(v):
                bits = plsc.bitcast(cand_s[pl.ds(v * 16, 16)], jnp.int32)
                bkt = lax.shift_right_logical(bits, 20)
                cnts, lastm = plsc.scan_count(bkt)
                plsc.addupdate_scatter(hist, [bkt], cnts, mask=lastm)

            def sv(v, carry):
                run, bv, runa = carry
                vv = NBV - 1 - v
                tv = jnp.sum(hist[pl.ds(vv * 16, 16)])
                found = (run < K) & (run + tv >= K)
                return (run + tv, jnp.where(found, vv, bv),
                        jnp.where(found, run, runa))

            _, bv, runa = plsc.parallel_loop(
                0, NBV, unroll=8,
                carry=(jnp.int32(0), jnp.int32(0), jnp.int32(0)))(sv)

            hv = hist[pl.ds(bv * 16, 16)]
            sincl = lax.rev(plsc.cumsum(lax.rev(hv, (0,))), (0,))
            sexcl = sincl - hv
            cond = ((runa + sexcl) < K) & ((runa + sincl) >= K)
            lstar = jnp.max(jnp.where(cond, lane, 0))
            count_above = jnp.max(jnp.where(cond, runa + sexcl, 0))
            subcnt = jnp.max(jnp.where(cond, hv, 0))
            bstar = bv * 16 + lstar
            kp = K - count_above
            lo0 = lax.shift_left(bstar, 20)
            hi0 = lo0 + (1 << 20)

            def subset_path(_):
                def scp(v, off2):
                    s = cand_s[pl.ds(v * 16, 16)]
                    m2 = lax.shift_right_logical(
                        plsc.bitcast(s, jnp.int32), 20) == bstar
                    plsc.store_compressed(cand2.at[pl.ds(off2, 16)], s,
                                          mask=m2)
                    return off2 + jnp.sum(m2.astype(jnp.int32))

                off2 = lax.fori_loop(0, nv, scp, jnp.int32(0))
                cand2[pl.ds(off2, 16)] = zf
                nv2 = off2 // 16 + 1

                def bs2(it, lohi):
                    lo, hi = lohi
                    mid = (lo + hi + 1) // 2

                    def cb2(v, acc):
                        bits = plsc.bitcast(cand2[pl.ds(v * 16, 16)],
                                            jnp.int32)
                        return acc + jnp.where(bits >= mid, 1, 0)

                    cnt = jnp.sum(plsc.parallel_loop(0, nv2, unroll=2,
                                                     carry=zi)(cb2))
                    ok = cnt >= kp
                    return (jnp.where(ok, mid, lo), jnp.where(ok, hi, mid))

                t, _ = lax.fori_loop(0, 20, bs2, (lo0, hi0))
                return t

            def full_path(_):
                def bs3(it, lohi):
                    lo, hi = lohi
                    mid = (lo + hi + 1) // 2

                    def cb3(v, acc):
                        bits = plsc.bitcast(cand_s[pl.ds(v * 16, 16)],
                                            jnp.int32)
                        return acc + jnp.where(bits >= mid, 1, 0)

                    cnt = jnp.sum(plsc.parallel_loop(0, nv, unroll=8,
                                                     carry=zi)(cb3))
                    ok = cnt >= K
                    return (jnp.where(ok, mid, lo), jnp.where(ok, hi, mid))

                t, _ = lax.fori_loop(0, 20, bs3, (lo0, hi0))
                return t

            tau = lax.cond(subcnt <= 512, subset_path, full_path,
                           jnp.int32(0))

            def p_c(v, carry):
                ssum, wsum = carry
                s = cand_s[pl.ds(v * 16, 16)]
                w = cand_w[pl.ds(v * 16, 16)]
                sel = plsc.bitcast(s, jnp.int32) >= tau
                return (ssum + jnp.where(sel, s, 0.0),
                        wsum + jnp.where(sel, s * w, 0.0))

            ssum, wsum = plsc.parallel_loop(0, nv, unroll=4,
                                            carry=(zf, zf))(p_c)
            return jnp.max((zf + jnp.sum(wsum))
                           / (zf + jnp.sum(ssum) + 1e-8))

        def chunk_body(c, carry):
            cb = c * CH
            h1 = pltpu.async_copy(isim_hbm.at[iid_v.at[pl.ds(cb, CH)]],
                                  sim_buf, sem)
            h2 = pltpu.async_copy(qrows_hbm.at[idx_qrow.at[pl.ds(cb, CH)]],
                                  qrow_buf, sem)
            h3 = pltpu.async_copy(qcols_hbm.at[idx_qcol.at[pl.ds(cb, CH)]],
                                  col_buf, sem)
            h4 = pltpu.async_copy(usim_hbm.at[uid_v.at[pl.ds(cb, CH)]],
                                  usim_buf, sem)
            h1.wait()
            h2.wait()
            h3.wait()
            h4.wait()

            def row_body(r, _rc):
                s_pos = cb + r
                spl = jnp.zeros((16,), jnp.int32) + s_pos
                t_s = jnp.max(plsc.load_gather(tid_v, [spl]))
                u_s = jnp.max(plsc.load_gather(uid_v, [spl]))
                i_s = jnp.max(plsc.load_gather(iid_v, [spl]))
                ib = t_s * I
                ub = t_s * U
                p_i = branch(
                    lambda j: sim_buf[r, pl.ds(j * 16, 16)],
                    lambda j: qrow_buf[r, pl.ds(j * 16, 16)],
                    lambda j: iavg_v[pl.ds(ib + j * 16, 16)], I)
                p_u = branch(
                    lambda j: usim_buf[r, pl.ds(j * 16, 16)],
                    lambda j: col_buf[r, pl.ds(j * 16, 16)],
                    lambda j: uavg_v[pl.ds(ub + j * 16, 16)], U)
                rspl = jnp.zeros((16,), jnp.int32) + r
                curr = jnp.max(plsc.load_gather(qrow_buf,
                                                [rspl, zi + i_s]))
                ts_s = jnp.max(plsc.load_gather(ts_v, [spl]))
                tc_s = jnp.max(plsc.load_gather(tc_v, [spl]))
                nz = jnp.where(curr > 0.0, 1.0, 0.0)
                sum_o = ts_s - curr
                cnt_o = tc_s - nz
                qt = jnp.max((zf + sum_o) / (zf + jnp.maximum(cnt_o, 1.0)))
                p_t = jnp.where(cnt_o > 0.0, qt, 0.0)
                bi_s = jnp.max(plsc.load_gather(iavg_v, [zi + (ib + i_s)]))
                bu_s = jnp.max(plsc.load_gather(uavg_v, [zi + (ub + u_s)]))
                val = w0 * p_t + w1 * (bu_s + p_u) + w2 * (bi_s + p_i)
                plsc.store_scatter(out_v, [spl], zf + val, mask=(lane == 0))
                return _rc

            lax.fori_loop(0, CH, row_body, jnp.int32(0))
            return carry

        lax.fori_loop(0, NCH, chunk_body, jnp.int32(0))
        pltpu.sync_copy(out_v, out_hbm.at[pl.ds(base, BPW)])

    return sc_kernel


def kernel(qos, item_avg, user_avg, item_sim_agg, user_sim_agg,
           total_sum, total_cnt, weights, user_id, item_id, time_id):
    T, U, I = qos.shape
    B = user_id.shape[0]

    qrows = qos.reshape(T * U, I)
    qcols = qos.transpose(0, 2, 1).reshape(T * I, U)
    tsg = total_sum.reshape(U * I // 16, 16)
    tcg = total_cnt.reshape(U * I // 16, 16)
    iavg_flat = item_avg.reshape(T * I)
    uavg_flat = user_avg.reshape(T * U)
    w_pad = jnp.zeros((16,), jnp.float32).at[:3].set(weights)

    sc = _make_sc_kernel(T, U, I, B)
    return sc(qrows, qcols, item_sim_agg, user_sim_agg, tsg, tcg,
              iavg_flat, uavg_flat, w_pad, user_id, item_id, time_id)


# speedup vs baseline: 37.6668x; 1.1855x over previous
import functools

import jax
import jax.numpy as jnp
from jax import lax
from jax.experimental import pallas as pl
from jax.experimental.pallas import tpu as pltpu
from jax.experimental.pallas import tpu_sc as plsc

K = 50
NW = 32
CH = 8


def _make_sc_kernel(T, U, I, B):
    BPW = B // NW
    NCH = BPW // CH
    NG = BPW // 16

    mesh = plsc.VectorSubcoreMesh(core_axis_name="c", subcore_axis_name="s")

    scratch = [
        pltpu.VMEM((BPW,), jnp.int32),
        pltpu.VMEM((BPW,), jnp.int32),
        pltpu.VMEM((BPW,), jnp.int32),
        pltpu.VMEM((BPW,), jnp.int32),
        pltpu.VMEM((BPW,), jnp.int32),
        pltpu.VMEM((BPW,), jnp.int32),
        pltpu.VMEM((32, 128), jnp.float32),
        pltpu.VMEM((BPW,), jnp.float32),
        pltpu.VMEM((BPW,), jnp.float32),
        pltpu.VMEM((BPW,), jnp.float32),
        pltpu.VMEM((T * I,), jnp.float32),
        pltpu.VMEM((T * U,), jnp.float32),
        pltpu.VMEM((16,), jnp.float32),
        pltpu.VMEM((CH, I), jnp.float32),
        pltpu.VMEM((CH, I), jnp.float32),
        pltpu.VMEM((CH, U), jnp.float32),
        pltpu.VMEM((CH, U), jnp.float32),
        pltpu.VMEM((I + 16,), jnp.float32),
        pltpu.VMEM((I + 16,), jnp.float32),
        pltpu.VMEM((1024,), jnp.int32),
        pltpu.VMEM((528,), jnp.float32),
        pltpu.SemaphoreType.DMA,
    ]

    @functools.partial(
        pl.kernel, mesh=mesh,
        out_type=jax.ShapeDtypeStruct((B,), jnp.float32),
        compiler_params=pltpu.CompilerParams(needs_layout_passes=False,
                                             use_tc_tiling_on_sc=True),
        scratch_types=scratch,
    )
    def sc_kernel(qrows_hbm, qcols_hbm, isim_hbm, usim_hbm,
                  tsg_hbm, tcg_hbm,
                  iavg_hbm, uavg_hbm, w_hbm, uid_hbm, iid_hbm, tid_hbm,
                  out_hbm,
                  uid_v, iid_v, tid_v,
                  idx_qrow, idx_qcol, idx_ts,
                  tmp_g, ts_v, tc_v, out_v,
                  iavg_v, uavg_v, w_v,
                  sim_buf, qrow_buf, col_buf, usim_buf,
                  cand_s, cand_w, hist, cand2, sem):
        wid = lax.axis_index("s") * 2 + lax.axis_index("c")
        base = wid * BPW

        pltpu.sync_copy(uid_hbm.at[pl.ds(base, BPW)], uid_v)
        pltpu.sync_copy(iid_hbm.at[pl.ds(base, BPW)], iid_v)
        pltpu.sync_copy(tid_hbm.at[pl.ds(base, BPW)], tid_v)
        pltpu.sync_copy(iavg_hbm, iavg_v)
        pltpu.sync_copy(uavg_hbm, uavg_v)
        pltpu.sync_copy(w_hbm, w_v)

        lane = lax.iota(jnp.int32, 16)

        for g in range(NG):
            sl = pl.ds(g * 16, 16)
            u = uid_v[sl]
            i = iid_v[sl]
            t = tid_v[sl]
            idx_qrow[sl] = t * U + u
            idx_qcol[sl] = t * I + i
            idx_ts[sl] = (u * I + i) // 128

        for tbl, outr in ((tsg_hbm, ts_v), (tcg_hbm, tc_v)):
            for sw in range(BPW // 32):
                pltpu.async_copy(tbl.at[idx_ts.at[pl.ds(sw * 32, 32)]],
                                 tmp_g, sem).wait()
                for g in range(2):
                    sl = pl.ds(sw * 32 + g * 16, 16)
                    u = uid_v[sl]
                    i = iid_v[sl]
                    rows = lane + g * 16
                    outr[sl] = plsc.load_gather(
                        tmp_g, [rows, (u * I + i) % 128])

        w16 = w_v[pl.ds(0, 16)]
        w0 = jnp.sum(jnp.where(lane == 0, w16, 0.0))
        w1 = jnp.sum(jnp.where(lane == 1, w16, 0.0))
        w2 = jnp.sum(jnp.where(lane == 2, w16, 0.0))

        zf = jnp.zeros((16,), jnp.float32)
        zi = jnp.zeros((16,), jnp.int32)

        def branch(load_sim, load_q, load_avg, n):
            def p_a(j, off):
                sim = load_sim(j)
                q = load_q(j)
                av = load_avg(j)
                m = (q > 0.0) & (sim > 0.0)
                plsc.store_compressed(cand_s.at[pl.ds(off, 16)], sim, mask=m)
                plsc.store_compressed(cand_w.at[pl.ds(off, 16)], q - av,
                                      mask=m)
                return off + jnp.sum(m.astype(jnp.int32))

            off = lax.fori_loop(0, n // 16, p_a, jnp.int32(0), unroll=4)
            cand_s[pl.ds(off, 16)] = zf
            cand_w[pl.ds(off, 16)] = zf
            nv = off // 16 + 1

            NBV = 64

            @plsc.parallel_loop(0, NBV, unroll=8)
            def _clear(v):
                hist[pl.ds(v * 16, 16)] = zi

            @plsc.parallel_loop(0, nv, unroll=4)
            def _hist(v):
                bits = plsc.bitcast(cand_s[pl.ds(v * 16, 16)], jnp.int32)
                bkt = lax.shift_right_logical(bits, 20)
                cnts, lastm = plsc.scan_count(bkt)
                plsc.addupdate_scatter(hist, [bkt], cnts, mask=lastm)

            def sv(v, carry):
                run, bv, runa = carry
                vv = NBV - 1 - v
                tv = jnp.sum(hist[pl.ds(vv * 16, 16)])
                found = (run < K) & (run + tv >= K)
                return (run + tv, jnp.where(found, vv, bv),
                        jnp.where(found, run, runa))

            _, bv, runa = plsc.parallel_loop(
                0, NBV, unroll=8,
                carry=(jnp.int32(0), jnp.int32(0), jnp.int32(0)))(sv)

            hv = hist[pl.ds(bv * 16, 16)]
            sincl = lax.rev(plsc.cumsum(lax.rev(hv, (0,))), (0,))
            sexcl = sincl - hv
            cond = ((runa + sexcl) < K) & ((runa + sincl) >= K)
            lstar = jnp.max(jnp.where(cond, lane, 0))
            count_above = jnp.max(jnp.where(cond, runa + sexcl, 0))
            subcnt = jnp.max(jnp.where(cond, hv, 0))
            bstar = bv * 16 + lstar
            kp = K - count_above
            lo0 = lax.shift_left(bstar, 20)
            hi0 = lo0 + (1 << 20)

            def subset_path(_):
                def scp(v, off2):
                    s = cand_s[pl.ds(v * 16, 16)]
                    m2 = lax.shift_right_logical(
                        plsc.bitcast(s, jnp.int32), 20) == bstar
                    plsc.store_compressed(cand2.at[pl.ds(off2, 16)], s,
                                          mask=m2)
                    return off2 + jnp.sum(m2.astype(jnp.int32))

                off2 = lax.fori_loop(0, nv, scp, jnp.int32(0))
                cand2[pl.ds(off2, 16)] = zf
                nv2 = off2 // 16 + 1

                def bs2(it, lohi):
                    lo, hi = lohi
                    mid = (lo + hi + 1) // 2

                    def cb2(v, acc):
                        bits = plsc.bitcast(cand2[pl.ds(v * 16, 16)],
                                            jnp.int32)
                        return acc + jnp.where(bits >= mid, 1, 0)

                    cnt = jnp.sum(plsc.parallel_loop(0, nv2, unroll=2,
                                                     carry=zi)(cb2))
                    ok = cnt >= kp
                    return (jnp.where(ok, mid, lo), jnp.where(ok, hi, mid))

                t, _ = lax.fori_loop(0, 20, bs2, (lo0, hi0))
                return t

            def full_path(_):
                def bs3(it, lohi):
                    lo, hi = lohi
                    mid = (lo + hi + 1) // 2

                    def cb3(v, acc):
                        bits = plsc.bitcast(cand_s[pl.ds(v * 16, 16)],
                                            jnp.int32)
                        return acc + jnp.where(bits >= mid, 1, 0)

                    cnt = jnp.sum(plsc.parallel_loop(0, nv, unroll=8,
                                                     carry=zi)(cb3))
                    ok = cnt >= K
                    return (jnp.where(ok, mid, lo), jnp.where(ok, hi, mid))

                t, _ = lax.fori_loop(0, 20, bs3, (lo0, hi0))
                return t

            tau = lax.cond(subcnt <= 512, subset_path, full_path,
                           jnp.int32(0))

            def p_c(v, carry):
                ssum, wsum = carry
                s = cand_s[pl.ds(v * 16, 16)]
                w = cand_w[pl.ds(v * 16, 16)]
                sel = plsc.bitcast(s, jnp.int32) >= tau
                return (ssum + jnp.where(sel, s, 0.0),
                        wsum + jnp.where(sel, s * w, 0.0))

            ssum, wsum = plsc.parallel_loop(0, nv, unroll=4,
                                            carry=(zf, zf))(p_c)
            return jnp.max((zf + jnp.sum(wsum))
                           / (zf + jnp.sum(ssum) + 1e-8))

        def chunk_body(c, carry):
            cb = c * CH
            h1 = pltpu.async_copy(isim_hbm.at[iid_v.at[pl.ds(cb, CH)]],
                                  sim_buf, sem)
            h2 = pltpu.async_copy(qrows_hbm.at[idx_qrow.at[pl.ds(cb, CH)]],
                                  qrow_buf, sem)
            h3 = pltpu.async_copy(qcols_hbm.at[idx_qcol.at[pl.ds(cb, CH)]],
                                  col_buf, sem)
            h4 = pltpu.async_copy(usim_hbm.at[uid_v.at[pl.ds(cb, CH)]],
                                  usim_buf, sem)
            h1.wait()
            h2.wait()
            h3.wait()
            h4.wait()

            def row_body(r, _rc):
                s_pos = cb + r
                spl = jnp.zeros((16,), jnp.int32) + s_pos
                t_s = jnp.max(plsc.load_gather(tid_v, [spl]))
                u_s = jnp.max(plsc.load_gather(uid_v, [spl]))
                i_s = jnp.max(plsc.load_gather(iid_v, [spl]))
                ib = t_s * I
                ub = t_s * U
                p_i = branch(
                    lambda j: sim_buf[r, pl.ds(j * 16, 16)],
                    lambda j: qrow_buf[r, pl.ds(j * 16, 16)],
                    lambda j: iavg_v[pl.ds(ib + j * 16, 16)], I)
                p_u = branch(
                    lambda j: usim_buf[r, pl.ds(j * 16, 16)],
                    lambda j: col_buf[r, pl.ds(j * 16, 16)],
                    lambda j: uavg_v[pl.ds(ub + j * 16, 16)], U)
                rspl = jnp.zeros((16,), jnp.int32) + r
                curr = jnp.max(plsc.load_gather(qrow_buf,
                                                [rspl, zi + i_s]))
                ts_s = jnp.max(plsc.load_gather(ts_v, [spl]))
                tc_s = jnp.max(plsc.load_gather(tc_v, [spl]))
                nz = jnp.where(curr > 0.0, 1.0, 0.0)
                sum_o = ts_s - curr
                cnt_o = tc_s - nz
                qt = jnp.max((zf + sum_o) / (zf + jnp.maximum(cnt_o, 1.0)))
                p_t = jnp.where(cnt_o > 0.0, qt, 0.0)
                bi_s = jnp.max(plsc.load_gather(iavg_v, [zi + (ib + i_s)]))
                bu_s = jnp.max(plsc.load_gather(uavg_v, [zi + (ub + u_s)]))
                val = w0 * p_t + w1 * (bu_s + p_u) + w2 * (bi_s + p_i)
                plsc.store_scatter(out_v, [spl], zf + val, mask=(lane == 0))
                return _rc

            lax.fori_loop(0, CH, row_body, jnp.int32(0))
            return carry

        lax.fori_loop(0, NCH, chunk_body, jnp.int32(0))
        pltpu.sync_copy(out_v, out_hbm.at[pl.ds(base, BPW)])

    return sc_kernel


def kernel(qos, item_avg, user_avg, item_sim_agg, user_sim_agg,
           total_sum, total_cnt, weights, user_id, item_id, time_id):
    T, U, I = qos.shape
    B = user_id.shape[0]

    qrows = qos.reshape(T * U, I)
    qcols = qos.transpose(0, 2, 1).reshape(T * I, U)
    tsg = total_sum.reshape(U * I // 128, 128)
    tcg = total_cnt.reshape(U * I // 128, 128)
    iavg_flat = item_avg.reshape(T * I)
    uavg_flat = user_avg.reshape(T * U)
    w_pad = jnp.zeros((16,), jnp.float32).at[:3].set(weights)

    sc = _make_sc_kernel(T, U, I, B)
    return sc(qrows, qcols, item_sim_agg, user_sim_agg, tsg, tcg,
              iavg_flat, uavg_flat, w_pad, user_id, item_id, time_id)
